# baseline (speedup 1.0000x reference)
import functools

import jax
import jax.numpy as jnp
from jax import lax
from jax.experimental import pallas as pl
from jax.experimental.pallas import tpu as pltpu
from jax.experimental.pallas import tpu_sc as plsc

D = 1024
H = 4096
M = H // 2
OUT = 1024
E = 8
K = 2
T = 2048

BM = 256
NB = (K * T) // BM + E
PADMAX = NB * BM

NW = 32
CH = 16


def _gate_body(x_ref, wg1_ref, bg1_ref, wg2_ref, bg2_ref, posw_ref, meta_ref):
    f32 = jnp.float32
    hp = None
    xx = x_ref[...]
    gh = jnp.maximum(jnp.dot(xx, wg1_ref[...], precision=hp) + bg1_ref[...], 0.0)
    logits = jnp.dot(gh, wg2_ref[...], precision=hp) + bg2_ref[...]
    mx = jnp.max(logits, axis=-1, keepdims=True)
    ex = jnp.exp(logits - mx)
    gw = ex / jnp.sum(ex, axis=-1, keepdims=True)

    lane = lax.broadcasted_iota(jnp.int32, (T, 128), 1).astype(f32)
    lane_valid = lane < E
    usage = jnp.sum(gw, axis=0, keepdims=True) / T
    loss = jnp.sum(jnp.where(lane_valid[:1], (usage - 1.0 / E) ** 2, 0.0)) / E

    gwm = jnp.where(lane_valid, gw, -1.0)
    m1 = jnp.max(gwm, axis=-1, keepdims=True)
    i1 = jnp.min(jnp.where(gwm == m1, lane, 1e9), axis=-1, keepdims=True)
    gw2 = jnp.where(lane == i1, -2.0, gwm)
    m2 = jnp.max(gw2, axis=-1, keepdims=True)
    i2 = jnp.min(jnp.where(gw2 == m2, lane, 1e9), axis=-1, keepdims=True)

    e21 = jnp.exp(m2 - m1)
    w0 = 1.0 / (1.0 + e21)
    w1 = e21 * w0

    oh0 = (lane == i1).astype(f32)
    oh1 = (lane == i2).astype(f32)
    iota_r = lax.broadcasted_iota(jnp.int32, (T, T), 0)
    iota_c = lax.broadcasted_iota(jnp.int32, (T, T), 1)
    tri = (iota_r > iota_c).astype(f32)
    cum0 = jnp.dot(tri, oh0, preferred_element_type=f32)
    cum1 = jnp.dot(tri, oh1, preferred_element_type=f32)
    cnt0 = jnp.sum(oh0, axis=0, keepdims=True)
    cnt = cnt0 + jnp.sum(oh1, axis=0, keepdims=True)

    nblk = jnp.floor((cnt + (BM - 1)) * (1.0 / BM))
    padded = nblk * BM
    er = lax.broadcasted_iota(jnp.int32, (128, 128), 0).astype(f32)
    ec = lax.broadcasted_iota(jnp.int32, (128, 128), 1).astype(f32)
    lt = (er < ec).astype(f32)
    off = jnp.dot(padded, lt, preferred_element_type=f32)

    rank0 = jnp.sum(cum0 * oh0, axis=-1, keepdims=True)
    rank1 = jnp.sum((cum1 + cnt0) * oh1, axis=-1, keepdims=True)
    base0 = jnp.sum(off * oh0, axis=-1, keepdims=True)
    base1 = jnp.sum(off * oh1, axis=-1, keepdims=True)
    pos0 = base0 + rank0
    pos1 = base1 + rank1

    eye = (er == ec).astype(f32)
    offcol = jnp.sum(off * eye, axis=-1, keepdims=True)
    padcol = jnp.sum(padded * eye, axis=-1, keepdims=True)
    bs = ec * BM
    cond = jnp.logical_and(offcol <= bs, bs < offcol + padcol)
    cond = jnp.logical_and(cond, er < E)
    condf = cond.astype(f32)
    be = jnp.sum(condf * er, axis=0, keepdims=True)
    be = jnp.where(jnp.sum(condf, axis=0, keepdims=True) > 0, be, -1.0)
    meta_ref[...] = jnp.broadcast_to(be, (8, 128)).astype(jnp.int32)

    posw = (pos0 * (lane == 0) + pos1 * (lane == 1) + w0 * (lane == 2)
            + w1 * (lane == 3) + loss * (lane == 4))
    posw_ref[...] = posw


def _gate(x, wg1, bg1p, wg2p, bg2p):
    return pl.pallas_call(
        _gate_body,
        out_shape=(
            jax.ShapeDtypeStruct((T, 128), jnp.float32),
            jax.ShapeDtypeStruct((8, 128), jnp.int32),
        ),
    )(x, wg1, bg1p, wg2p, bg2p)


SCB = T // 16


def _scatter_body(x_hbm, pos_hbm, xs_hbm, rows_v, pos_v, sem):
    wid = lax.axis_index("s") * 2 + lax.axis_index("c")
    k = wid // 16
    t0 = (wid % 16) * SCB
    pltpu.sync_copy(x_hbm.at[pl.ds(t0, SCB), :], rows_v)
    pltpu.sync_copy(pos_hbm.at[k, pl.ds(t0, SCB)], pos_v)
    pltpu.async_copy(rows_v, xs_hbm.at[pos_v], sem).wait()


@functools.cache
def _scatter_x_kernel():
    return pl.kernel(
        _scatter_body,
        mesh=plsc.VectorSubcoreMesh(core_axis_name="c", subcore_axis_name="s"),
        out_type=jax.ShapeDtypeStruct((PADMAX, D // 2), jnp.int32),
        scratch_types=[
            pltpu.VMEM((SCB, D // 2), jnp.int32),
            pltpu.VMEM((SCB,), jnp.int32),
            pltpu.SemaphoreType.DMA,
        ],
    )


def _scatter_x(x, pos):
    return _scatter_x_kernel()(x, pos)


def _mlp_body(be_ref, xs_ref, w_ref, b_ref, o_ref, *, act):
    i = pl.program_id(0)

    @pl.when(be_ref[i] >= 0)
    def _():
        acc = jnp.dot(xs_ref[...], w_ref[0],
                      preferred_element_type=jnp.float32)
        acc = acc + b_ref[0]
        if act:
            acc = jnp.maximum(acc, 0.0)
        o_ref[...] = acc.astype(o_ref.dtype)


def _grouped_mlp_layer(be, xs, w, b3d, act, out_dtype):
    _, kdim, n = w.shape
    grid_spec = pltpu.PrefetchScalarGridSpec(
        num_scalar_prefetch=1,
        grid=(NB,),
        in_specs=[
            pl.BlockSpec((BM, kdim), lambda i, be_ref: (i, 0)),
            pl.BlockSpec((1, kdim, n),
                         lambda i, be_ref: (jnp.maximum(be_ref[i], 0), 0, 0)),
            pl.BlockSpec((1, 1, n),
                         lambda i, be_ref: (jnp.maximum(be_ref[i], 0), 0, 0)),
        ],
        out_specs=pl.BlockSpec((BM, n), lambda i, be_ref: (i, 0)),
    )
    return pl.pallas_call(
        functools.partial(_mlp_body, act=act),
        grid_spec=grid_spec,
        out_shape=jax.ShapeDtypeStruct((PADMAX, n), out_dtype),
    )(be, xs, w, b3d)


CMB = 32


def _combine_body(ys_hbm, pos_hbm, wb_hbm, out_hbm,
                  r0_v, r1_v, o_v, p0_v, p1_v, w0_v, w1_v, sem):
    wid = lax.axis_index("s") * 2 + lax.axis_index("c")
    tpw = T // NW
    for c in range(tpw // CMB):
        t0 = wid * tpw + c * CMB
        pltpu.sync_copy(pos_hbm.at[0, pl.ds(t0, CMB)], p0_v)
        pltpu.sync_copy(pos_hbm.at[1, pl.ds(t0, CMB)], p1_v)
        cp0 = pltpu.async_copy(ys_hbm.at[p0_v], r0_v, sem)
        cp1 = pltpu.async_copy(ys_hbm.at[p1_v], r1_v, sem)
        pltpu.sync_copy(wb_hbm.at[0, pl.ds(t0, CMB), :], w0_v)
        pltpu.sync_copy(wb_hbm.at[1, pl.ds(t0, CMB), :], w1_v)
        cp0.wait()
        cp1.wait()
        for r in range(CMB):
            wr0 = w0_v[r, :]
            wr1 = w1_v[r, :]

            def dchunk(j, _, wr0=wr0, wr1=wr1, r=r):
                sl = pl.ds(j * 16, 16)
                o_v[r, sl] = wr0 * r0_v[r, sl] + wr1 * r1_v[r, sl]
                return 0

            lax.fori_loop(0, OUT // 16, dchunk, 0)
        pltpu.sync_copy(o_v, out_hbm.at[pl.ds(t0, CMB), :])


@functools.cache
def _combine_kernel():
    return pl.kernel(
        _combine_body,
        mesh=plsc.VectorSubcoreMesh(core_axis_name="c", subcore_axis_name="s"),
        out_type=jax.ShapeDtypeStruct((T, OUT), jnp.float32),
        scratch_types=[
            pltpu.VMEM((CMB, OUT), jnp.float32),
            pltpu.VMEM((CMB, OUT), jnp.float32),
            pltpu.VMEM((CMB, OUT), jnp.float32),
            pltpu.VMEM((CMB,), jnp.int32),
            pltpu.VMEM((CMB,), jnp.int32),
            pltpu.VMEM((CMB, 16), jnp.float32),
            pltpu.VMEM((CMB, 16), jnp.float32),
            pltpu.SemaphoreType.DMA,
        ],
    )


def _combine(ys, pos, wb):
    return _combine_kernel()(ys, pos, wb)


def kernel(x, W1, b1, W2, b2, W3, b3, Wg1, bg1, Wg2, bg2):
    f32 = jnp.float32
    bg1p = bg1.reshape(1, D // 2)
    wg2p = jnp.pad(Wg2, ((0, 0), (0, 128 - E)))
    bg2p = jnp.pad(bg2, (0, 128 - E), constant_values=-1e30).reshape(1, 128)

    posw, meta = _gate(x, Wg1, bg1p, wg2p, bg2p)
    pos = posw[:, :K].T.astype(jnp.int32)
    wb = jnp.broadcast_to(posw[:, K:2 * K].T[:, :, None], (K, T, 16))
    wb = jnp.asarray(wb, f32)
    loss = posw[0, 2 * K]
    be = meta[0, :NB]

    bf16 = jnp.bfloat16
    x32 = lax.bitcast_convert_type(x.astype(bf16).reshape(T, D // 2, 2),
                                   jnp.int32)
    xs32 = _scatter_x(x32, pos)
    xs = lax.bitcast_convert_type(xs32, bf16).reshape(PADMAX, D)
    h1 = _grouped_mlp_layer(be, xs, W1.astype(bf16), b1.reshape(E, 1, H),
                            True, bf16)
    h2 = _grouped_mlp_layer(be, h1, W2.astype(bf16), b2.reshape(E, 1, M),
                            True, bf16)
    ys = _grouped_mlp_layer(be, h2, W3.astype(bf16), b3.reshape(E, 1, OUT),
                            False, f32)

    out = _combine(ys, pos, wb)
    return out, loss


# speedup vs baseline: 1.1639x; 1.1639x over previous
import functools

import jax
import jax.numpy as jnp
from jax import lax
from jax.experimental import pallas as pl
from jax.experimental.pallas import tpu as pltpu
from jax.experimental.pallas import tpu_sc as plsc

D = 1024
H = 4096
M = H // 2
OUT = 1024
E = 8
K = 2
T = 2048

BM = 256
NB = (K * T) // BM + E
PADMAX = NB * BM

NW = 32
CH = 16


def _gate_body(x_ref, wg1_ref, bg1_ref, wg2_ref, bg2_ref, posw_ref, meta_ref):
    f32 = jnp.float32
    hp = None
    xx = x_ref[...]
    gh = jnp.maximum(jnp.dot(xx, wg1_ref[...], precision=hp) + bg1_ref[...], 0.0)
    logits = jnp.dot(gh, wg2_ref[...], precision=hp) + bg2_ref[...]
    mx = jnp.max(logits, axis=-1, keepdims=True)
    ex = jnp.exp(logits - mx)
    gw = ex / jnp.sum(ex, axis=-1, keepdims=True)

    lane = lax.broadcasted_iota(jnp.int32, (T, 128), 1).astype(f32)
    lane_valid = lane < E
    usage = jnp.sum(gw, axis=0, keepdims=True) / T
    loss = jnp.sum(jnp.where(lane_valid[:1], (usage - 1.0 / E) ** 2, 0.0)) / E

    gwm = jnp.where(lane_valid, gw, -1.0)
    m1 = jnp.max(gwm, axis=-1, keepdims=True)
    i1 = jnp.min(jnp.where(gwm == m1, lane, 1e9), axis=-1, keepdims=True)
    gw2 = jnp.where(lane == i1, -2.0, gwm)
    m2 = jnp.max(gw2, axis=-1, keepdims=True)
    i2 = jnp.min(jnp.where(gw2 == m2, lane, 1e9), axis=-1, keepdims=True)

    e21 = jnp.exp(m2 - m1)
    w0 = 1.0 / (1.0 + e21)
    w1 = e21 * w0

    oh0 = (lane == i1).astype(f32)
    oh1 = (lane == i2).astype(f32)
    iota_r = lax.broadcasted_iota(jnp.int32, (T, T), 0)
    iota_c = lax.broadcasted_iota(jnp.int32, (T, T), 1)
    tri = (iota_r > iota_c).astype(f32)
    cum0 = jnp.dot(tri, oh0, preferred_element_type=f32)
    cum1 = jnp.dot(tri, oh1, preferred_element_type=f32)
    cnt0 = jnp.sum(oh0, axis=0, keepdims=True)
    cnt = cnt0 + jnp.sum(oh1, axis=0, keepdims=True)

    nblk = jnp.floor((cnt + (BM - 1)) * (1.0 / BM))
    padded = nblk * BM
    er = lax.broadcasted_iota(jnp.int32, (128, 128), 0).astype(f32)
    ec = lax.broadcasted_iota(jnp.int32, (128, 128), 1).astype(f32)
    lt = (er < ec).astype(f32)
    off = jnp.dot(padded, lt, preferred_element_type=f32)

    rank0 = jnp.sum(cum0 * oh0, axis=-1, keepdims=True)
    rank1 = jnp.sum((cum1 + cnt0) * oh1, axis=-1, keepdims=True)
    base0 = jnp.sum(off * oh0, axis=-1, keepdims=True)
    base1 = jnp.sum(off * oh1, axis=-1, keepdims=True)
    pos0 = base0 + rank0
    pos1 = base1 + rank1

    eye = (er == ec).astype(f32)
    offcol = jnp.sum(off * eye, axis=-1, keepdims=True)
    padcol = jnp.sum(padded * eye, axis=-1, keepdims=True)
    bs = ec * BM
    cond = jnp.logical_and(offcol <= bs, bs < offcol + padcol)
    cond = jnp.logical_and(cond, er < E)
    condf = cond.astype(f32)
    be = jnp.sum(condf * er, axis=0, keepdims=True)
    be = jnp.where(jnp.sum(condf, axis=0, keepdims=True) > 0, be, -1.0)
    meta_ref[...] = jnp.broadcast_to(be, (8, 128)).astype(jnp.int32)

    posw = (pos0 * (lane == 0) + pos1 * (lane == 1) + w0 * (lane == 2)
            + w1 * (lane == 3) + loss * (lane == 4))
    posw_ref[...] = posw


def _gate(x, wg1, bg1p, wg2p, bg2p):
    return pl.pallas_call(
        _gate_body,
        out_shape=(
            jax.ShapeDtypeStruct((T, 128), jnp.float32),
            jax.ShapeDtypeStruct((8, 128), jnp.int32),
        ),
    )(x, wg1, bg1p, wg2p, bg2p)


SCB = T // 16


def _scatter_body(x_hbm, pos_hbm, xs_hbm, rows_v, pos_v, sem):
    wid = lax.axis_index("s") * 2 + lax.axis_index("c")
    k = wid // 16
    t0 = (wid % 16) * SCB
    pltpu.sync_copy(x_hbm.at[pl.ds(t0, SCB), :], rows_v)
    pltpu.sync_copy(pos_hbm.at[k, pl.ds(t0, SCB)], pos_v)
    pltpu.async_copy(rows_v, xs_hbm.at[pos_v], sem).wait()


@functools.cache
def _scatter_x_kernel():
    return pl.kernel(
        _scatter_body,
        mesh=plsc.VectorSubcoreMesh(core_axis_name="c", subcore_axis_name="s"),
        out_type=jax.ShapeDtypeStruct((PADMAX, D // 2), jnp.int32),
        scratch_types=[
            pltpu.VMEM((SCB, D // 2), jnp.int32),
            pltpu.VMEM((SCB,), jnp.int32),
            pltpu.SemaphoreType.DMA,
        ],
    )


def _scatter_x(x, pos):
    return _scatter_x_kernel()(x, pos)


def _mlp_body(be_ref, xs_ref, w_ref, b_ref, o_ref, *, act):
    i = pl.program_id(1)

    @pl.when(be_ref[i] >= 0)
    def _():
        acc = jnp.dot(xs_ref[...], w_ref[0].astype(jnp.bfloat16),
                      preferred_element_type=jnp.float32)
        acc = acc + b_ref[0]
        if act:
            acc = jnp.maximum(acc, 0.0)
        o_ref[...] = acc.astype(o_ref.dtype)


def _grouped_mlp_layer(be, xs, w, b3d, act, out_dtype, tn):
    _, kdim, n = w.shape
    nt = n // tn
    grid_spec = pltpu.PrefetchScalarGridSpec(
        num_scalar_prefetch=1,
        grid=(nt, NB),
        in_specs=[
            pl.BlockSpec((BM, kdim), lambda j, i, be_ref: (i, 0)),
            pl.BlockSpec((1, kdim, tn),
                         lambda j, i, be_ref: (jnp.maximum(be_ref[i], 0),
                                               0, j)),
            pl.BlockSpec((1, 1, tn),
                         lambda j, i, be_ref: (jnp.maximum(be_ref[i], 0),
                                               0, j)),
        ],
        out_specs=pl.BlockSpec((BM, tn), lambda j, i, be_ref: (i, j)),
    )
    return pl.pallas_call(
        functools.partial(_mlp_body, act=act),
        grid_spec=grid_spec,
        out_shape=jax.ShapeDtypeStruct((PADMAX, n), out_dtype),
    )(be, xs, w, b3d)


CMB = 32


def _combine_body(ys_hbm, pos_hbm, wb_hbm, out_hbm,
                  r0_v, r1_v, o_v, p0_v, p1_v, w0_v, w1_v, sem):
    wid = lax.axis_index("s") * 2 + lax.axis_index("c")
    tpw = T // NW
    for c in range(tpw // CMB):
        t0 = wid * tpw + c * CMB
        pltpu.sync_copy(pos_hbm.at[0, pl.ds(t0, CMB)], p0_v)
        pltpu.sync_copy(pos_hbm.at[1, pl.ds(t0, CMB)], p1_v)
        cp0 = pltpu.async_copy(ys_hbm.at[p0_v], r0_v, sem)
        cp1 = pltpu.async_copy(ys_hbm.at[p1_v], r1_v, sem)
        pltpu.sync_copy(wb_hbm.at[0, pl.ds(t0, CMB), :], w0_v)
        pltpu.sync_copy(wb_hbm.at[1, pl.ds(t0, CMB), :], w1_v)
        cp0.wait()
        cp1.wait()
        for r in range(CMB):
            wr0 = w0_v[r, :]
            wr1 = w1_v[r, :]

            def dchunk(j, _, wr0=wr0, wr1=wr1, r=r):
                sl = pl.ds(j * 16, 16)
                o_v[r, sl] = wr0 * r0_v[r, sl] + wr1 * r1_v[r, sl]
                return 0

            lax.fori_loop(0, OUT // 16, dchunk, 0)
        pltpu.sync_copy(o_v, out_hbm.at[pl.ds(t0, CMB), :])


@functools.cache
def _combine_kernel():
    return pl.kernel(
        _combine_body,
        mesh=plsc.VectorSubcoreMesh(core_axis_name="c", subcore_axis_name="s"),
        out_type=jax.ShapeDtypeStruct((T, OUT), jnp.float32),
        scratch_types=[
            pltpu.VMEM((CMB, OUT), jnp.float32),
            pltpu.VMEM((CMB, OUT), jnp.float32),
            pltpu.VMEM((CMB, OUT), jnp.float32),
            pltpu.VMEM((CMB,), jnp.int32),
            pltpu.VMEM((CMB,), jnp.int32),
            pltpu.VMEM((CMB, 16), jnp.float32),
            pltpu.VMEM((CMB, 16), jnp.float32),
            pltpu.SemaphoreType.DMA,
        ],
    )


def _combine(ys, pos, wb):
    return _combine_kernel()(ys, pos, wb)


def kernel(x, W1, b1, W2, b2, W3, b3, Wg1, bg1, Wg2, bg2):
    f32 = jnp.float32
    bg1p = bg1.reshape(1, D // 2)
    wg2p = jnp.pad(Wg2, ((0, 0), (0, 128 - E)))
    bg2p = jnp.pad(bg2, (0, 128 - E), constant_values=-1e30).reshape(1, 128)

    posw, meta = _gate(x, Wg1, bg1p, wg2p, bg2p)
    pos = posw[:, :K].T.astype(jnp.int32)
    wb = jnp.broadcast_to(posw[:, K:2 * K].T[:, :, None], (K, T, 16))
    wb = jnp.asarray(wb, f32)
    loss = posw[0, 2 * K]
    be = meta[0, :NB]

    bf16 = jnp.bfloat16
    x32 = lax.bitcast_convert_type(x.astype(bf16).reshape(T, D // 2, 2),
                                   jnp.int32)
    xs32 = _scatter_x(x32, pos)
    xs = lax.bitcast_convert_type(xs32, bf16).reshape(PADMAX, D)
    h1 = _grouped_mlp_layer(be, xs, W1, b1.reshape(E, 1, H), True, bf16, 2048)
    h2 = _grouped_mlp_layer(be, h1, W2, b2.reshape(E, 1, M), True, bf16, 1024)
    ys = _grouped_mlp_layer(be, h2, W3, b3.reshape(E, 1, OUT), False, f32,
                            1024)

    out = _combine(ys, pos, wb)
    return out, loss


# speedup vs baseline: 1.1789x; 1.0129x over previous
import functools

import jax
import jax.numpy as jnp
from jax import lax
from jax.experimental import pallas as pl
from jax.experimental.pallas import tpu as pltpu
from jax.experimental.pallas import tpu_sc as plsc

D = 1024
H = 4096
M = H // 2
OUT = 1024
E = 8
K = 2
T = 2048

BM = 256
NB = (K * T) // BM + E
PADMAX = NB * BM

NW = 32
CH = 16


def _gate_body(x_ref, wg1_ref, bg1_ref, wg2_ref, bg2_ref, posw_ref, meta_ref):
    f32 = jnp.float32
    hp = None
    xx = x_ref[...]
    gh = jnp.maximum(jnp.dot(xx, wg1_ref[...], precision=hp) + bg1_ref[...], 0.0)
    logits = jnp.dot(gh, wg2_ref[...], precision=hp) + bg2_ref[...]
    mx = jnp.max(logits, axis=-1, keepdims=True)
    ex = jnp.exp(logits - mx)
    gw = ex / jnp.sum(ex, axis=-1, keepdims=True)

    lane = lax.broadcasted_iota(jnp.int32, (T, 128), 1).astype(f32)
    lane_valid = lane < E
    usage = jnp.sum(gw, axis=0, keepdims=True) / T
    loss = jnp.sum(jnp.where(lane_valid[:1], (usage - 1.0 / E) ** 2, 0.0)) / E

    gwm = jnp.where(lane_valid, gw, -1.0)
    m1 = jnp.max(gwm, axis=-1, keepdims=True)
    i1 = jnp.min(jnp.where(gwm == m1, lane, 1e9), axis=-1, keepdims=True)
    gw2 = jnp.where(lane == i1, -2.0, gwm)
    m2 = jnp.max(gw2, axis=-1, keepdims=True)
    i2 = jnp.min(jnp.where(gw2 == m2, lane, 1e9), axis=-1, keepdims=True)

    e21 = jnp.exp(m2 - m1)
    w0 = 1.0 / (1.0 + e21)
    w1 = e21 * w0

    bf16 = jnp.bfloat16
    oh0 = (lane == i1).astype(f32)
    oh1 = (lane == i2).astype(f32)
    iota_r = lax.broadcasted_iota(jnp.int32, (T, T), 0)
    iota_c = lax.broadcasted_iota(jnp.int32, (T, T), 1)
    tri = (iota_r > iota_c).astype(bf16)
    cum0 = jnp.dot(tri, oh0.astype(bf16), preferred_element_type=f32)
    cum1 = jnp.dot(tri, oh1.astype(bf16), preferred_element_type=f32)
    cnt0 = jnp.sum(oh0, axis=0, keepdims=True)
    cnt = cnt0 + jnp.sum(oh1, axis=0, keepdims=True)

    nblk = jnp.floor((cnt + (BM - 1)) * (1.0 / BM))
    padded = nblk * BM
    er = lax.broadcasted_iota(jnp.int32, (128, 128), 0).astype(f32)
    ec = lax.broadcasted_iota(jnp.int32, (128, 128), 1).astype(f32)
    lt = (er < ec).astype(f32)
    off = jnp.dot(padded, lt, preferred_element_type=f32)

    rank0 = jnp.sum(cum0 * oh0, axis=-1, keepdims=True)
    rank1 = jnp.sum((cum1 + cnt0) * oh1, axis=-1, keepdims=True)
    base0 = jnp.sum(off * oh0, axis=-1, keepdims=True)
    base1 = jnp.sum(off * oh1, axis=-1, keepdims=True)
    pos0 = base0 + rank0
    pos1 = base1 + rank1

    eye = (er == ec).astype(f32)
    offcol = jnp.sum(off * eye, axis=-1, keepdims=True)
    padcol = jnp.sum(padded * eye, axis=-1, keepdims=True)
    bs = ec * BM
    cond = jnp.logical_and(offcol <= bs, bs < offcol + padcol)
    cond = jnp.logical_and(cond, er < E)
    condf = cond.astype(f32)
    be = jnp.sum(condf * er, axis=0, keepdims=True)
    be = jnp.where(jnp.sum(condf, axis=0, keepdims=True) > 0, be, -1.0)
    meta_ref[...] = jnp.broadcast_to(be, (8, 128)).astype(jnp.int32)

    posw = (pos0 * (lane == 0) + pos1 * (lane == 1) + w0 * (lane == 2)
            + w1 * (lane == 3) + loss * (lane == 4))
    posw_ref[...] = posw


def _gate(x, wg1, bg1p, wg2p, bg2p):
    return pl.pallas_call(
        _gate_body,
        out_shape=(
            jax.ShapeDtypeStruct((T, 128), jnp.float32),
            jax.ShapeDtypeStruct((8, 128), jnp.int32),
        ),
    )(x, wg1, bg1p, wg2p, bg2p)


SCB = T // 16


def _scatter_body(x_hbm, pos_hbm, xs_hbm, rows_v, pos_v, sem):
    wid = lax.axis_index("s") * 2 + lax.axis_index("c")
    k = wid // 16
    t0 = (wid % 16) * SCB
    pltpu.sync_copy(x_hbm.at[pl.ds(t0, SCB), :], rows_v)
    pltpu.sync_copy(pos_hbm.at[k, pl.ds(t0, SCB)], pos_v)
    pltpu.async_copy(rows_v, xs_hbm.at[pos_v], sem).wait()


@functools.cache
def _scatter_x_kernel():
    return pl.kernel(
        _scatter_body,
        mesh=plsc.VectorSubcoreMesh(core_axis_name="c", subcore_axis_name="s"),
        out_type=jax.ShapeDtypeStruct((PADMAX, D // 2), jnp.int32),
        scratch_types=[
            pltpu.VMEM((SCB, D // 2), jnp.int32),
            pltpu.VMEM((SCB,), jnp.int32),
            pltpu.SemaphoreType.DMA,
        ],
    )


def _scatter_x(x, pos):
    return _scatter_x_kernel()(x, pos)


def _mlp_body(be_ref, xs_ref, w_ref, b_ref, o_ref, *, act):
    i = pl.program_id(1)

    @pl.when(be_ref[i] >= 0)
    def _():
        acc = jnp.dot(xs_ref[...], w_ref[0].astype(jnp.bfloat16),
                      preferred_element_type=jnp.float32)
        acc = acc + b_ref[0]
        if act:
            acc = jnp.maximum(acc, 0.0)
        o_ref[...] = acc.astype(o_ref.dtype)


def _grouped_mlp_layer(be, xs, w, b3d, act, out_dtype, tn):
    _, kdim, n = w.shape
    nt = n // tn
    grid_spec = pltpu.PrefetchScalarGridSpec(
        num_scalar_prefetch=1,
        grid=(nt, NB),
        in_specs=[
            pl.BlockSpec((BM, kdim), lambda j, i, be_ref: (i, 0)),
            pl.BlockSpec((1, kdim, tn),
                         lambda j, i, be_ref: (jnp.maximum(be_ref[i], 0),
                                               0, j)),
            pl.BlockSpec((1, 1, tn),
                         lambda j, i, be_ref: (jnp.maximum(be_ref[i], 0),
                                               0, j)),
        ],
        out_specs=pl.BlockSpec((BM, tn), lambda j, i, be_ref: (i, j)),
    )
    return pl.pallas_call(
        functools.partial(_mlp_body, act=act),
        grid_spec=grid_spec,
        out_shape=jax.ShapeDtypeStruct((PADMAX, n), out_dtype),
    )(be, xs, w, b3d)


CMB = 32


def _gather_body(ys_hbm, pos_hbm, out_hbm, r0_v, r1_v, p0_v, p1_v, sem):
    wid = lax.axis_index("s") * 2 + lax.axis_index("c")
    tpw = T // NW
    for c in range(tpw // CMB):
        t0 = wid * tpw + c * CMB
        pltpu.sync_copy(pos_hbm.at[0, pl.ds(t0, CMB)], p0_v)
        pltpu.sync_copy(pos_hbm.at[1, pl.ds(t0, CMB)], p1_v)
        cp0 = pltpu.async_copy(ys_hbm.at[p0_v], r0_v, sem)
        cp1 = pltpu.async_copy(ys_hbm.at[p1_v], r1_v, sem)
        cp0.wait()
        cp1.wait()
        pltpu.sync_copy(r0_v, out_hbm.at[0, pl.ds(t0, CMB), :])
        pltpu.sync_copy(r1_v, out_hbm.at[1, pl.ds(t0, CMB), :])


@functools.cache
def _gather_kernel():
    return pl.kernel(
        _gather_body,
        mesh=plsc.VectorSubcoreMesh(core_axis_name="c", subcore_axis_name="s"),
        out_type=jax.ShapeDtypeStruct((K, T, OUT), jnp.float32),
        scratch_types=[
            pltpu.VMEM((CMB, OUT), jnp.float32),
            pltpu.VMEM((CMB, OUT), jnp.float32),
            pltpu.VMEM((CMB,), jnp.int32),
            pltpu.VMEM((CMB,), jnp.int32),
            pltpu.SemaphoreType.DMA,
        ],
    )


def _wadd_body(g_ref, posw_ref, o_ref):
    lane = lax.broadcasted_iota(jnp.int32, (T, 128), 1)
    pw = posw_ref[...]
    w0 = jnp.sum(jnp.where(lane == 2, pw, 0.0), axis=1, keepdims=True)
    w1 = jnp.sum(jnp.where(lane == 3, pw, 0.0), axis=1, keepdims=True)
    o_ref[...] = w0 * g_ref[0] + w1 * g_ref[1]


def _combine(ys, pos, posw):
    g = _gather_kernel()(ys, pos)
    return pl.pallas_call(
        _wadd_body,
        out_shape=jax.ShapeDtypeStruct((T, OUT), jnp.float32),
    )(g, posw)


def kernel(x, W1, b1, W2, b2, W3, b3, Wg1, bg1, Wg2, bg2):
    f32 = jnp.float32
    bg1p = bg1.reshape(1, D // 2)
    wg2p = jnp.pad(Wg2, ((0, 0), (0, 128 - E)))
    bg2p = jnp.pad(bg2, (0, 128 - E), constant_values=-1e30).reshape(1, 128)

    posw, meta = _gate(x, Wg1, bg1p, wg2p, bg2p)
    pos = posw[:, :K].T.astype(jnp.int32)
    loss = posw[0, 2 * K]
    be = meta[0, :NB]

    bf16 = jnp.bfloat16
    x32 = lax.bitcast_convert_type(x.astype(bf16).reshape(T, D // 2, 2),
                                   jnp.int32)
    xs32 = _scatter_x(x32, pos)
    xs = lax.bitcast_convert_type(xs32, bf16).reshape(PADMAX, D)
    h1 = _grouped_mlp_layer(be, xs, W1, b1.reshape(E, 1, H), True, bf16, 2048)
    h2 = _grouped_mlp_layer(be, h1, W2, b2.reshape(E, 1, M), True, bf16, 1024)
    ys = _grouped_mlp_layer(be, h2, W3, b3.reshape(E, 1, OUT), False, f32,
                            1024)

    out = _combine(ys, pos, posw)
    return out, loss
